# baseline (speedup 1.0000x reference)
import functools

import jax
import jax.numpy as jnp
from jax import lax
from jax.experimental import pallas as pl
from jax.experimental.pallas import tpu as pltpu
from jax.experimental.pallas import tpu_sc as plsc

_NC = 2
_NS = 16
_NW = _NC * _NS


@functools.lru_cache(maxsize=None)
def _make_gather(B: int, D: int, C: int):
    bpw = B // _NW
    nchunk = bpw // C
    mesh = plsc.VectorSubcoreMesh(core_axis_name="c", subcore_axis_name="s")

    @functools.partial(
        pl.kernel,
        mesh=mesh,
        out_type=jax.ShapeDtypeStruct((B, D), jnp.float32),
        scratch_types=[
            pltpu.VMEM((C,), jnp.int32),
            pltpu.VMEM((C, D), jnp.float32),
            pltpu.SemaphoreType.DMA,
        ],
        compiler_params=pltpu.CompilerParams(use_tc_tiling_on_sc=False),
    )
    def emb_gather(idx_hbm, table_hbm, out_hbm, idx_v, rows_v, sem):
        wid = lax.axis_index("s") * _NC + lax.axis_index("c")
        base = wid * bpw

        def body(i, carry):
            off = base + i * C
            pltpu.sync_copy(idx_hbm.at[pl.ds(off, C)], idx_v)
            pltpu.async_copy(table_hbm.at[idx_v], rows_v, sem).wait()
            pltpu.sync_copy(rows_v, out_hbm.at[pl.ds(off, C)])
            return carry

        lax.fori_loop(0, nchunk, body, 0)

    return emb_gather


def kernel(x, emb_loc):
    B = x.size
    D = emb_loc.shape[1]
    xf = x.reshape(-1).astype(jnp.int32)
    out = _make_gather(B, D, 512)(xf, emb_loc)
    return out.reshape(x.shape + (D,))


# speedup vs baseline: 1.0422x; 1.0422x over previous
import functools

import jax
import jax.numpy as jnp
from jax import lax
from jax.experimental import pallas as pl
from jax.experimental.pallas import tpu as pltpu
from jax.experimental.pallas import tpu_sc as plsc

_NC = 2
_NS = 16
_NW = _NC * _NS


@functools.lru_cache(maxsize=None)
def _make_gather(B: int, D: int, C: int, NBUF: int):
    bpw = B // _NW
    nchunk = bpw // C
    ngrp = nchunk // NBUF
    assert bpw % C == 0 and nchunk % NBUF == 0 and ngrp >= 2
    mesh = plsc.VectorSubcoreMesh(core_axis_name="c", subcore_axis_name="s")

    @functools.partial(
        pl.kernel,
        mesh=mesh,
        out_type=jax.ShapeDtypeStruct((B, D), jnp.float32),
        scratch_types=[
            pltpu.VMEM((bpw,), jnp.int32),
            pltpu.VMEM((NBUF, C, D), jnp.float32),
        ]
        + [pltpu.SemaphoreType.DMA] * (2 * NBUF),
        compiler_params=pltpu.CompilerParams(use_tc_tiling_on_sc=False),
    )
    def emb_gather(idx_hbm, table_hbm, out_hbm, idx_v, rows_v, *sems):
        gsem = sems[:NBUF]
        ssem = sems[NBUF:]
        wid = lax.axis_index("s") * _NC + lax.axis_index("c")
        base = wid * bpw

        pltpu.sync_copy(idx_hbm.at[pl.ds(base, bpw)], idx_v)

        def start_gather(chunk, b):
            pltpu.async_copy(
                table_hbm.at[idx_v.at[pl.ds(chunk * C, C)]], rows_v.at[b], gsem[b]
            )

        def start_store(chunk, b):
            pltpu.async_copy(
                rows_v.at[b], out_hbm.at[pl.ds(base + chunk * C, C)], ssem[b]
            )

        for b in range(NBUF):
            start_gather(b, b)

        def group(g, carry):
            c0 = g * NBUF
            for b in range(NBUF):
                pltpu.make_async_copy(rows_v.at[b], out_hbm.at[pl.ds(0, C)],
                                      gsem[b]).wait()
                start_store(c0 + b, b)
            for b in range(NBUF):
                pltpu.make_async_copy(rows_v.at[b], out_hbm.at[pl.ds(0, C)],
                                      ssem[b]).wait()
                start_gather(c0 + NBUF + b, b)
            return carry

        lax.fori_loop(0, ngrp - 1, group, 0)

        c0 = (ngrp - 1) * NBUF
        for b in range(NBUF):
            pltpu.make_async_copy(rows_v.at[b], out_hbm.at[pl.ds(0, C)],
                                  gsem[b]).wait()
            start_store(c0 + b, b)
        for b in range(NBUF):
            pltpu.make_async_copy(rows_v.at[b], out_hbm.at[pl.ds(0, C)],
                                  ssem[b]).wait()

    return emb_gather


def kernel(x, emb_loc):
    B = x.size
    D = emb_loc.shape[1]
    xf = x.reshape(-1).astype(jnp.int32)
    out = _make_gather(B, D, 512, 2)(xf, emb_loc)
    return out.reshape(x.shape + (D,))


# speedup vs baseline: 1.0460x; 1.0037x over previous
import functools

import jax
import jax.numpy as jnp
from jax import lax
from jax.experimental import pallas as pl
from jax.experimental.pallas import tpu as pltpu
from jax.experimental.pallas import tpu_sc as plsc

_NC = 2
_NS = 16
_NW = _NC * _NS


@functools.lru_cache(maxsize=None)
def _make_gather(B: int, D: int, C: int, NBUF: int):
    SL = NBUF // 2
    bpw = B // _NW
    nchunk = bpw // C
    ngrp = nchunk // NBUF
    assert bpw % C == 0 and nchunk % NBUF == 0 and ngrp >= 2
    mesh = plsc.VectorSubcoreMesh(core_axis_name="c", subcore_axis_name="s")

    @functools.partial(
        pl.kernel,
        mesh=mesh,
        out_type=jax.ShapeDtypeStruct((B, D), jnp.float32),
        scratch_types=[
            pltpu.VMEM((bpw,), jnp.int32),
            pltpu.VMEM((NBUF, C, D), jnp.float32),
        ]
        + [pltpu.SemaphoreType.DMA] * (2 * NBUF),
        compiler_params=pltpu.CompilerParams(use_tc_tiling_on_sc=False),
    )
    def emb_gather(idx_hbm, table_hbm, out_hbm, idx_v, rows_v, *sems):
        gsem = sems[:NBUF]
        ssem = sems[NBUF:]
        wid = lax.axis_index("s") * _NC + lax.axis_index("c")
        base = wid * bpw

        pltpu.sync_copy(idx_hbm.at[pl.ds(base, bpw)], idx_v)

        def start_gather(chunk, b):
            pltpu.async_copy(
                table_hbm.at[idx_v.at[pl.ds(chunk * C, C)]], rows_v.at[b], gsem[b]
            )

        def start_store(chunk, b):
            pltpu.async_copy(
                rows_v.at[b], out_hbm.at[pl.ds(base + chunk * C, C)], ssem[b]
            )

        def wait_g(b):
            pltpu.make_async_copy(rows_v.at[b], out_hbm.at[pl.ds(0, C)],
                                  gsem[b]).wait()

        def wait_s(b):
            pltpu.make_async_copy(rows_v.at[b], out_hbm.at[pl.ds(0, C)],
                                  ssem[b]).wait()

        for b in range(NBUF):
            start_gather(b, b)
            if b >= SL:
                wait_g(b - SL)
                start_store(b - SL, b - SL)

        def group(g, carry):
            c0 = g * NBUF
            for b in range(NBUF):
                wait_s(b)
                start_gather(c0 + b, b)
                b2 = (b - SL) % NBUF
                wait_g(b2)
                start_store(c0 + b - SL, b2)
            return carry

        lax.fori_loop(1, ngrp, group, 0)

        for k in range(SL):
            i = nchunk - SL + k
            b = i % NBUF
            wait_g(b)
            start_store(i, b)
        for b in range(NBUF):
            wait_s(b)

    return emb_gather


def kernel(x, emb_loc):
    B = x.size
    D = emb_loc.shape[1]
    xf = x.reshape(-1).astype(jnp.int32)
    out = _make_gather(B, D, 320, 4)(xf, emb_loc)
    return out.reshape(x.shape + (D,))
